# Initial kernel scaffold; baseline (speedup 1.0000x reference)
import jax
import jax.numpy as jnp
from jax.experimental import pallas as pl


def kernel(x, edge_index):
    raise NotImplementedError("write your pallas kernel here")


# speedup vs baseline: 703.0578x; 703.0578x over previous
import jax
import jax.numpy as jnp
import numpy as np
from jax import lax
from jax.experimental import pallas as pl
from jax.experimental.pallas import tpu as pltpu
from jax.experimental.pallas import tpu_sc as plsc

_N = 10000
_NP = 10240
_SLICE = _NP // 16
_NSUB = 16
_E = 320000
_EPT = _E // _NSUB
_CHUNK = 128
_NCH = 160
_EPAD = _NCH * _CHUNK
_PAD = _NP - 1
_ALPHA = 0.85
_MAXIT = 100
_THRESH = np.float32(_N * 1e-06)
_V0 = np.float32(1.0 / _N)
_ADDEND = np.float32(np.float32(1.0 / _N) * np.float32(1.0 - _ALPHA))
_LN = 16


def _body(rows_hbm, cols_hbm, out_hbm, rows_v, cols_v, ev,
          vsl, usl, msl, ainv, addend, zbuf, ones, errw, errall,
          u_sh, mv_sh, err_sh):
    s = lax.axis_index("s")
    c = lax.axis_index("c")
    base = s * _SLICE
    ebase = s * _NCH

    pltpu.sync_copy(rows_hbm.at[pl.ds(ebase, _NCH)], rows_v)
    pltpu.sync_copy(cols_hbm.at[pl.ds(ebase, _NCH)], cols_v)

    lane = lax.iota(jnp.int32, 16)
    for k in range(_SLICE // _LN):
        gidx = base + (k * _LN) + lane
        m = gidx < _N
        vsl[pl.ds(k * 16, 16)] = jnp.where(m, _V0, np.float32(0.0))
        addend[pl.ds(k * 16, 16)] = jnp.where(m, _ADDEND, np.float32(0.0))
        zbuf[pl.ds(k * 16, 16)] = jnp.zeros((16,), jnp.float32)
    for k in range(_CHUNK // _LN):
        ones[pl.ds(k * 16, 16)] = jnp.ones((16,), jnp.float32)

    pltpu.sync_copy(zbuf, mv_sh.at[pl.ds(base, _SLICE)])
    plsc.subcore_barrier()

    def degstep(j, carry):
        pltpu.sync_copy(ones, mv_sh.at[cols_v.at[j]], add=True)
        return carry

    lax.fori_loop(0, _NCH, degstep, 0)
    plsc.subcore_barrier()
    pltpu.sync_copy(mv_sh.at[pl.ds(base, _SLICE)], msl)
    for k in range(_SLICE // _LN):
        d = msl[pl.ds(k * 16, 16)]
        ainv[pl.ds(k * 16, 16)] = jnp.where(
            d > np.float32(0.0), np.float32(_ALPHA) / d, np.float32(0.0))
    pltpu.sync_copy(zbuf, mv_sh.at[pl.ds(base, _SLICE)])

    def itbody(i, done):
        for k in range(_SLICE // _LN):
            usl[pl.ds(k * 16, 16)] = (
                vsl[pl.ds(k * 16, 16)] * ainv[pl.ds(k * 16, 16)])
        pltpu.sync_copy(usl, u_sh.at[pl.ds(base, _SLICE)])
        plsc.subcore_barrier()

        def chunkstep(j, carry2):
            pltpu.sync_copy(u_sh.at[cols_v.at[j]], ev)
            pltpu.sync_copy(ev, mv_sh.at[rows_v.at[j]], add=True)
            return carry2

        nch = jnp.where(done, jnp.int32(0), jnp.int32(_NCH))
        lax.fori_loop(0, nch, chunkstep, 0)
        plsc.subcore_barrier()

        pltpu.sync_copy(mv_sh.at[pl.ds(base, _SLICE)], msl)
        pltpu.sync_copy(zbuf, mv_sh.at[pl.ds(base, _SLICE)])

        errv = jnp.zeros((16,), jnp.float32)
        for k in range(_SLICE // _LN):
            vold = vsl[pl.ds(k * 16, 16)]
            vn = msl[pl.ds(k * 16, 16)] + addend[pl.ds(k * 16, 16)]
            errv = errv + jnp.abs(vn - vold)
            vsl[pl.ds(k * 16, 16)] = jnp.where(done, vold, vn)
        errw[...] = errv
        pltpu.sync_copy(errw, err_sh.at[s])
        plsc.subcore_barrier()

        pltpu.sync_copy(err_sh, errall)
        tot = jnp.zeros((16,), jnp.float32)
        for k in range(_NSUB):
            tot = tot + errall[k]
        total = np.float32(0.0)
        for j in range(_LN):
            total = total + tot[j]
        return jnp.logical_or(done, total < _THRESH)

    lax.fori_loop(0, _MAXIT, itbody, jnp.bool_(False))

    @pl.when(c == jnp.int32(0))
    def _():
        pltpu.sync_copy(vsl, out_hbm.at[pl.ds(base, _SLICE)])


_pr_call = pl.kernel(
    _body,
    out_type=jax.ShapeDtypeStruct((_NP,), jnp.float32),
    mesh=plsc.VectorSubcoreMesh(
        core_axis_name="c", subcore_axis_name="s",
        num_cores=2, num_subcores=_NSUB),
    scratch_types=[
        pltpu.VMEM((_NCH, _CHUNK), jnp.int32),
        pltpu.VMEM((_NCH, _CHUNK), jnp.int32),
        pltpu.VMEM((_CHUNK,), jnp.float32),
        pltpu.VMEM((_SLICE,), jnp.float32),
        pltpu.VMEM((_SLICE,), jnp.float32),
        pltpu.VMEM((_SLICE,), jnp.float32),
        pltpu.VMEM((_SLICE,), jnp.float32),
        pltpu.VMEM((_SLICE,), jnp.float32),
        pltpu.VMEM((_SLICE,), jnp.float32),
        pltpu.VMEM((_CHUNK,), jnp.float32),
        pltpu.VMEM((16,), jnp.float32),
        pltpu.VMEM((16, 16), jnp.float32),
        pltpu.VMEM_SHARED((_NP,), jnp.float32),
        pltpu.VMEM_SHARED((_NP,), jnp.float32),
        pltpu.VMEM_SHARED((16, 16), jnp.float32),
    ],
)


@jax.jit
def kernel(x, edge_index):
    del x
    rows = edge_index[0]
    cols = edge_index[1]
    pad = jnp.full((_NSUB, _EPAD - _EPT), _PAD, jnp.int32)
    rows2d = jnp.concatenate(
        [rows.reshape(_NSUB, _EPT), pad], axis=1).reshape(_NSUB * _NCH, _CHUNK)
    cols2d = jnp.concatenate(
        [cols.reshape(_NSUB, _EPT), pad], axis=1).reshape(_NSUB * _NCH, _CHUNK)
    out = _pr_call(rows2d, cols2d)
    return out[:_N]


# speedup vs baseline: 1393.5343x; 1.9821x over previous
import jax
import jax.numpy as jnp
import numpy as np
from jax import lax
from jax.experimental import pallas as pl
from jax.experimental.pallas import tpu as pltpu
from jax.experimental.pallas import tpu_sc as plsc

_N = 10000
_NP = 10240
_SLICE = _NP // 16
_NSUB = 16
_E = 320000
_EPT = _E // _NSUB
_CHUNK = 128
_NCH = 160
_EPAD = _NCH * _CHUNK
_PAD = _NP - 1
_ALPHA = 0.85
_MAXIT = 100
_THRESH = np.float32(_N * 1e-06)
_V0 = np.float32(1.0 / _N)
_ADDEND = np.float32(np.float32(1.0 / _N) * np.float32(1.0 - _ALPHA))
_LN = 16


_G = 8
_NGRP = _NCH // _G


def _body(rows_hbm, cols_hbm, out_hbm, rows_v, cols_v, ev,
          vsl, usl, msl, ainv, addend, zbuf, ones, errw, errall,
          u_sh, mv_sh, err_sh, gsem, ssem):
    s = lax.axis_index("s")
    c = lax.axis_index("c")
    base = s * _SLICE
    ebase = s * _NCH

    pltpu.sync_copy(rows_hbm.at[pl.ds(ebase, _NCH)], rows_v)
    pltpu.sync_copy(cols_hbm.at[pl.ds(ebase, _NCH)], cols_v)

    lane = lax.iota(jnp.int32, 16)
    for k in range(_SLICE // _LN):
        gidx = base + (k * _LN) + lane
        m = gidx < _N
        vsl[pl.ds(k * 16, 16)] = jnp.where(m, _V0, np.float32(0.0))
        addend[pl.ds(k * 16, 16)] = jnp.where(m, _ADDEND, np.float32(0.0))
        zbuf[pl.ds(k * 16, 16)] = jnp.zeros((16,), jnp.float32)
    for k in range(_CHUNK // _LN):
        ones[pl.ds(k * 16, 16)] = jnp.ones((16,), jnp.float32)

    pltpu.sync_copy(zbuf, mv_sh.at[pl.ds(base, _SLICE)])
    plsc.subcore_barrier()

    def degstep(j, carry):
        pltpu.sync_copy(ones, mv_sh.at[cols_v.at[j]], add=True)
        return carry

    lax.fori_loop(0, _NCH, degstep, 0)
    plsc.subcore_barrier()
    pltpu.sync_copy(mv_sh.at[pl.ds(base, _SLICE)], msl)
    for k in range(_SLICE // _LN):
        d = msl[pl.ds(k * 16, 16)]
        ainv[pl.ds(k * 16, 16)] = jnp.where(
            d > np.float32(0.0), np.float32(_ALPHA) / d, np.float32(0.0))
    pltpu.sync_copy(zbuf, mv_sh.at[pl.ds(base, _SLICE)])
    errw[...] = jnp.ones((16,), jnp.float32)

    def _fire_gathers(g):
        hb = lax.rem(g, jnp.int32(2)) * _G
        for b in range(_G):
            pltpu.async_copy(u_sh.at[cols_v.at[g * _G + b]], ev.at[hb + b],
                             gsem)

    def _wait_gathers(g):
        hb = lax.rem(g, jnp.int32(2)) * _G
        for b in range(_G):
            pltpu.make_async_copy(u_sh.at[cols_v.at[g * _G + b]],
                                  ev.at[hb + b], gsem).wait()

    def _fire_scatters(g):
        hb = lax.rem(g, jnp.int32(2)) * _G
        for b in range(_G):
            pltpu.async_copy(ev.at[hb + b], mv_sh.at[rows_v.at[g * _G + b]],
                             ssem, add=True)

    def _wait_scatters(g):
        hb = lax.rem(g, jnp.int32(2)) * _G
        for b in range(_G):
            pltpu.make_async_copy(ev.at[hb + b],
                                  mv_sh.at[rows_v.at[g * _G + b]],
                                  ssem).wait()

    def itbody(i, _):
        t = errw[...]
        done = t[0] < _THRESH

        @pl.when(jnp.logical_not(done))
        def _():
            for k in range(_SLICE // _LN):
                usl[pl.ds(k * 16, 16)] = (
                    vsl[pl.ds(k * 16, 16)] * ainv[pl.ds(k * 16, 16)])
            pltpu.sync_copy(usl, u_sh.at[pl.ds(base, _SLICE)])
            plsc.subcore_barrier()

            _fire_gathers(jnp.int32(0))

            def groupstep(g, carry2):
                _wait_gathers(g)
                _fire_gathers(g + 1)
                _fire_scatters(g)
                _wait_scatters(g)
                return carry2

            lax.fori_loop(0, _NGRP - 1, groupstep, 0)
            g_last = jnp.int32(_NGRP - 1)
            _wait_gathers(g_last)
            _fire_scatters(g_last)
            _wait_scatters(g_last)
            plsc.subcore_barrier()

            pltpu.sync_copy(mv_sh.at[pl.ds(base, _SLICE)], msl)
            pltpu.sync_copy(zbuf, mv_sh.at[pl.ds(base, _SLICE)])

            errv = jnp.zeros((16,), jnp.float32)
            for k in range(_SLICE // _LN):
                vn = msl[pl.ds(k * 16, 16)] + addend[pl.ds(k * 16, 16)]
                errv = errv + jnp.abs(vn - vsl[pl.ds(k * 16, 16)])
                vsl[pl.ds(k * 16, 16)] = vn
            errw[...] = errv
            pltpu.sync_copy(errw, err_sh.at[s])
            plsc.subcore_barrier()

            pltpu.sync_copy(err_sh, errall)
            tot = jnp.zeros((16,), jnp.float32)
            for k in range(_NSUB):
                tot = tot + errall[k]
            total = np.float32(0.0)
            for j in range(_LN):
                total = total + tot[j]
            errw[...] = jnp.full((16,), np.float32(1.0)) * total

        return 0

    lax.fori_loop(0, _MAXIT, itbody, 0)

    @pl.when(c == jnp.int32(0))
    def _():
        pltpu.sync_copy(vsl, out_hbm.at[pl.ds(base, _SLICE)])


_pr_call = pl.kernel(
    _body,
    out_type=jax.ShapeDtypeStruct((_NP,), jnp.float32),
    mesh=plsc.VectorSubcoreMesh(
        core_axis_name="c", subcore_axis_name="s",
        num_cores=2, num_subcores=_NSUB),
    scratch_types=[
        pltpu.VMEM((_NCH, _CHUNK), jnp.int32),
        pltpu.VMEM((_NCH, _CHUNK), jnp.int32),
        pltpu.VMEM((2 * _G, _CHUNK), jnp.float32),
        pltpu.VMEM((_SLICE,), jnp.float32),
        pltpu.VMEM((_SLICE,), jnp.float32),
        pltpu.VMEM((_SLICE,), jnp.float32),
        pltpu.VMEM((_SLICE,), jnp.float32),
        pltpu.VMEM((_SLICE,), jnp.float32),
        pltpu.VMEM((_SLICE,), jnp.float32),
        pltpu.VMEM((_CHUNK,), jnp.float32),
        pltpu.VMEM((16,), jnp.float32),
        pltpu.VMEM((16, 16), jnp.float32),
        pltpu.VMEM_SHARED((_NP,), jnp.float32),
        pltpu.VMEM_SHARED((_NP,), jnp.float32),
        pltpu.VMEM_SHARED((16, 16), jnp.float32),
        pltpu.SemaphoreType.DMA,
        pltpu.SemaphoreType.DMA,
    ],
)


@jax.jit
def kernel(x, edge_index):
    del x
    rows = edge_index[0]
    cols = edge_index[1]
    pad = jnp.full((_NSUB, _EPAD - _EPT), _PAD, jnp.int32)
    rows2d = jnp.concatenate(
        [rows.reshape(_NSUB, _EPT), pad], axis=1).reshape(_NSUB * _NCH, _CHUNK)
    cols2d = jnp.concatenate(
        [cols.reshape(_NSUB, _EPT), pad], axis=1).reshape(_NSUB * _NCH, _CHUNK)
    out = _pr_call(rows2d, cols2d)
    return out[:_N]


# speedup vs baseline: 1434.8150x; 1.0296x over previous
import jax
import jax.numpy as jnp
import numpy as np
from jax import lax
from jax.experimental import pallas as pl
from jax.experimental.pallas import tpu as pltpu
from jax.experimental.pallas import tpu_sc as plsc

_N = 10000
_NP = 10240
_SLICE = _NP // 16
_NSUB = 16
_E = 320000
_EPT = _E // _NSUB
_EPAD = 20480
_PAD = _NP - 1
_ALPHA = 0.85
_MAXIT = 100
_THRESH = np.float32(_N * 1e-06)
_V0 = np.float32(1.0 / _N)
_ADDEND = np.float32(np.float32(1.0 / _N) * np.float32(1.0 - _ALPHA))
_LN = 16


def _body(rows_hbm, cols_hbm, out_hbm, rows_v, cols_v, ev,
          vsl, usl, msl, ainv, addend, zbuf, errw, errall,
          u_sh, mv_sh, err_sh):
    s = lax.axis_index("s")
    c = lax.axis_index("c")
    base = s * _SLICE
    ebase = s * _EPAD

    pltpu.sync_copy(rows_hbm.at[pl.ds(ebase, _EPAD)], rows_v)
    pltpu.sync_copy(cols_hbm.at[pl.ds(ebase, _EPAD)], cols_v)

    lane = lax.iota(jnp.int32, 16)
    for k in range(_SLICE // _LN):
        gidx = base + (k * _LN) + lane
        m = gidx < _N
        vsl[pl.ds(k * 16, 16)] = jnp.where(m, _V0, np.float32(0.0))
        addend[pl.ds(k * 16, 16)] = jnp.where(m, _ADDEND, np.float32(0.0))
        zbuf[pl.ds(k * 16, 16)] = jnp.zeros((16,), jnp.float32)
        usl[pl.ds(k * 16, 16)] = jnp.ones((16,), jnp.float32)

    pltpu.sync_copy(usl, u_sh.at[pl.ds(base, _SLICE)])
    pltpu.sync_copy(zbuf, mv_sh.at[pl.ds(base, _SLICE)])
    plsc.subcore_barrier()
    pltpu.sync_copy(u_sh.at[cols_v], ev)
    pltpu.sync_copy(ev, mv_sh.at[cols_v], add=True)
    plsc.subcore_barrier()
    pltpu.sync_copy(mv_sh.at[pl.ds(base, _SLICE)], msl)
    pltpu.sync_copy(zbuf, mv_sh.at[pl.ds(base, _SLICE)])
    for k in range(_SLICE // _LN):
        d = msl[pl.ds(k * 16, 16)]
        ainv[pl.ds(k * 16, 16)] = jnp.where(
            d > np.float32(0.0), np.float32(_ALPHA) / d, np.float32(0.0))

    errw[...] = jnp.ones((16,), jnp.float32)

    def itbody(i, _):
        t = errw[...]
        done = t[0] < _THRESH

        @pl.when(jnp.logical_not(done))
        def _():
            for k in range(_SLICE // _LN):
                usl[pl.ds(k * 16, 16)] = (
                    vsl[pl.ds(k * 16, 16)] * ainv[pl.ds(k * 16, 16)])
            pltpu.sync_copy(usl, u_sh.at[pl.ds(base, _SLICE)])
            plsc.subcore_barrier()

            pltpu.sync_copy(u_sh.at[cols_v], ev)
            pltpu.sync_copy(ev, mv_sh.at[rows_v], add=True)
            plsc.subcore_barrier()

            pltpu.sync_copy(mv_sh.at[pl.ds(base, _SLICE)], msl)
            pltpu.sync_copy(zbuf, mv_sh.at[pl.ds(base, _SLICE)])

            errv = jnp.zeros((16,), jnp.float32)
            for k in range(_SLICE // _LN):
                vn = msl[pl.ds(k * 16, 16)] + addend[pl.ds(k * 16, 16)]
                errv = errv + jnp.abs(vn - vsl[pl.ds(k * 16, 16)])
                vsl[pl.ds(k * 16, 16)] = vn
            errw[...] = errv
            pltpu.sync_copy(errw, err_sh.at[s])
            plsc.subcore_barrier()

            pltpu.sync_copy(err_sh, errall)
            tot = jnp.zeros((16,), jnp.float32)
            for k in range(_NSUB):
                tot = tot + errall[k]
            total = np.float32(0.0)
            for j in range(_LN):
                total = total + tot[j]
            errw[...] = jnp.full((16,), np.float32(1.0)) * total

        return 0

    lax.fori_loop(0, _MAXIT, itbody, 0)

    @pl.when(c == jnp.int32(0))
    def _():
        pltpu.sync_copy(vsl, out_hbm.at[pl.ds(base, _SLICE)])


_pr_call = pl.kernel(
    _body,
    out_type=jax.ShapeDtypeStruct((_NP,), jnp.float32),
    mesh=plsc.VectorSubcoreMesh(
        core_axis_name="c", subcore_axis_name="s",
        num_cores=2, num_subcores=_NSUB),
    scratch_types=[
        pltpu.VMEM((_EPAD,), jnp.int32),
        pltpu.VMEM((_EPAD,), jnp.int32),
        pltpu.VMEM((_EPAD,), jnp.float32),
        pltpu.VMEM((_SLICE,), jnp.float32),
        pltpu.VMEM((_SLICE,), jnp.float32),
        pltpu.VMEM((_SLICE,), jnp.float32),
        pltpu.VMEM((_SLICE,), jnp.float32),
        pltpu.VMEM((_SLICE,), jnp.float32),
        pltpu.VMEM((_SLICE,), jnp.float32),
        pltpu.VMEM((16,), jnp.float32),
        pltpu.VMEM((16, 16), jnp.float32),
        pltpu.VMEM_SHARED((_NP,), jnp.float32),
        pltpu.VMEM_SHARED((_NP,), jnp.float32),
        pltpu.VMEM_SHARED((16, 16), jnp.float32),
    ],
)


@jax.jit
def kernel(x, edge_index):
    del x
    rows = edge_index[0]
    cols = edge_index[1]
    pad = jnp.full((_NSUB, _EPAD - _EPT), _PAD, jnp.int32)
    rows1d = jnp.concatenate(
        [rows.reshape(_NSUB, _EPT), pad], axis=1).reshape(_NSUB * _EPAD)
    cols1d = jnp.concatenate(
        [cols.reshape(_NSUB, _EPT), pad], axis=1).reshape(_NSUB * _EPAD)
    out = _pr_call(rows1d, cols1d)
    return out[:_N]
